# baseline (speedup 1.0000x reference)
import functools

import jax
import jax.numpy as jnp
from jax import lax
from jax.experimental import pallas as pl
from jax.experimental.pallas import tpu as pltpu
from jax.experimental.pallas import tpu_sc as plsc

BATCH = 64
N = 4096
D = 128
NC = 2
NS = 16
NW = NC * NS
ROWS = BATCH * N
C = 128
NBUF = 4
NGROUP = 4
GBATCH = BATCH // NGROUP
GROWS = GBATCH * N


def _make_gather_body(rpw, nchunk, group_row0):
    def _gather_body(x_hbm, p_hbm, out_hbm, idx_v, *bufs):
        rows = bufs[:NBUF]
        gsem = bufs[NBUF : 2 * NBUF]
        wsem = bufs[2 * NBUF :]
        wid = lax.axis_index("s") * NC + lax.axis_index("c")
        base = wid * rpw
        pltpu.sync_copy(p_hbm.at[wid], idx_v)

        def add_off(k, carry):
            off = group_row0 + ((base + k * C) // N) * N
            for t in range(C // 16):
                sl = (k, pl.ds(t * 16, 16))
                idx_v[sl] = idx_v[sl] + off
            return carry

        lax.fori_loop(0, nchunk, add_off, 0)

        def start_gather(k, b):
            pltpu.async_copy(x_hbm.at[idx_v.at[k]], rows[b], gsem[b])

        def wait_gather(b):
            pltpu.make_async_copy(x_hbm.at[pl.ds(0, C)], rows[b], gsem[b]).wait()

        def start_write(k, b):
            pltpu.async_copy(rows[b], out_hbm.at[pl.ds(base + k * C, C)], wsem[b])

        def wait_write(b):
            pltpu.make_async_copy(rows[b], out_hbm.at[pl.ds(0, C)], wsem[b]).wait()

        for b in range(NBUF):
            start_gather(b, b)

        def ring(i, carry):
            for b in range(NBUF):
                k = i * NBUF + b
                wait_gather(b)
                start_write(k, b)
                nk = k + NBUF

                @pl.when(nk < nchunk)
                def _():
                    wait_write(b)
                    start_gather(nk, b)

            return carry

        lax.fori_loop(0, nchunk // NBUF, ring, 0)
        for b in range(NBUF):
            wait_write(b)

    return _gather_body


def _make_group_call(group):
    rpw = GROWS // NW
    nchunk = rpw // C
    mesh = plsc.VectorSubcoreMesh(
        core_axis_name="c", subcore_axis_name="s", num_cores=NC, num_subcores=NS
    )
    return pl.kernel(
        _make_gather_body(rpw, nchunk, group * GROWS),
        out_type=jax.ShapeDtypeStruct((GROWS, D), jnp.float32),
        mesh=mesh,
        scratch_types=(
            [pltpu.VMEM((nchunk, C), jnp.int32)]
            + [pltpu.VMEM((C, D), jnp.float32) for _ in range(NBUF)]
            + [pltpu.SemaphoreType.DMA for _ in range(2 * NBUF)]
        ),
        name=f"perm_gather_g{group}",
    )


_GROUP_CALLS = [_make_group_call(g) for g in range(NGROUP)]


def kernel(x, y):
    xf = x.reshape(ROWS, D)
    outs = []
    for g in range(NGROUP):
        yg = lax.slice_in_dim(y, g * GBATCH, (g + 1) * GBATCH)
        perm = jax.vmap(
            lambda yi: jax.random.permutation(jax.random.PRNGKey(yi), N)
        )(yg)
        p3 = perm.astype(jnp.int32).reshape(NW, GROWS // NW // C, C)
        outs.append(_GROUP_CALLS[g](xf, p3))
    return jnp.concatenate(outs, axis=0).reshape(BATCH, N, D)


# speedup vs baseline: 1.0365x; 1.0365x over previous
import functools

import jax
import jax.numpy as jnp
from jax import lax
from jax.experimental import pallas as pl
from jax.experimental.pallas import tpu as pltpu
from jax.experimental.pallas import tpu_sc as plsc

BATCH = 64
N = 4096
D = 128
NC = 2
NS = 16
NW = NC * NS
ROWS = BATCH * N
BPW = BATCH // NW
L = 16
VV = N // L
C = 128
NCHUNK = N // C
NBUF = 4


def _lane_iota():
    return lax.iota(jnp.int32, L)


def _digit(k, sh):
    if sh:
        k = lax.shift_right_logical(k, jnp.full((L,), sh, jnp.int32))
    return jnp.bitwise_and(k, jnp.full((L,), 255, jnp.int32))


def _radix_pass(ksrc, vsrc, kdst, vdst, hist, sh, first_round_pass0):
    lanes = _lane_iota()

    def zero(t, c):
        hist[pl.ds(t * L, L)] = jnp.zeros((L,), jnp.int32)
        return c

    lax.fori_loop(0, 256, zero, 0)

    def histo(v, c):
        idx = lanes * VV + v
        k = plsc.load_gather(ksrc, [idx])
        h = _digit(k, sh) * L + lanes
        plsc.addupdate_scatter(hist, [h], jnp.ones((L,), jnp.int32))
        return c

    lax.fori_loop(0, VV, histo, 0)

    def scan(t, running):
        h = hist[pl.ds(t * L, L)]
        s = plsc.cumsum(h)
        hist[pl.ds(t * L, L)] = s - h + running
        return running + jnp.sum(h)

    lax.fori_loop(0, 256, scan, jnp.int32(0))

    def scatter(v, c):
        idx = lanes * VV + v
        k = plsc.load_gather(ksrc, [idx])
        if first_round_pass0:
            val = idx
        else:
            val = plsc.load_gather(vsrc, [idx])
        h = _digit(k, sh) * L + lanes
        pos = plsc.load_gather(hist, [h])
        plsc.store_scatter(kdst, [pos], k)
        plsc.store_scatter(vdst, [pos], val)
        plsc.store_scatter(hist, [h], pos + 1)
        return c

    lax.fori_loop(0, VV, scatter, 0)


def _radix_argsort(ka, kb, va, vb, hist, first_round):
    _radix_pass(ka, va, kb, vb, hist, 0, first_round)
    _radix_pass(kb, vb, ka, va, hist, 8, False)
    _radix_pass(ka, va, kb, vb, hist, 16, False)
    _radix_pass(kb, vb, ka, va, hist, 24, False)


def _body(x_hbm, b1_hbm, b2_hbm, out_hbm, ka, kb, va, vb, hist, idx_v, *bufs):
    rows = bufs[:NBUF]
    gsem = bufs[NBUF : 2 * NBUF]
    wsem = bufs[2 * NBUF :]
    wid = lax.axis_index("s") * NC + lax.axis_index("c")

    def do_batch(bi, c):
        batch = wid * BPW + bi
        pltpu.sync_copy(b1_hbm.at[batch], ka)
        _radix_argsort(ka, kb, va, vb, hist, True)
        pltpu.sync_copy(b2_hbm.at[batch], ka)
        _radix_argsort(ka, kb, va, vb, hist, False)
        off = batch * N

        def mkidx(kc, c2):
            for t in range(C // L):
                sl = pl.ds(kc * C + t * L, L)
                idx_v[kc, pl.ds(t * L, L)] = va[sl] + off
            return c2

        lax.fori_loop(0, NCHUNK, mkidx, 0)

        obase = batch * N

        def start_gather(k, b):
            pltpu.async_copy(x_hbm.at[idx_v.at[k]], rows[b], gsem[b])

        def wait_gather(b):
            pltpu.make_async_copy(x_hbm.at[pl.ds(0, C)], rows[b], gsem[b]).wait()

        def start_write(k, b):
            pltpu.async_copy(rows[b], out_hbm.at[pl.ds(obase + k * C, C)], wsem[b])

        def wait_write(b):
            pltpu.make_async_copy(rows[b], out_hbm.at[pl.ds(0, C)], wsem[b]).wait()

        for b in range(NBUF):
            start_gather(b, b)

        def ring(i, c2):
            for b in range(NBUF):
                k = i * NBUF + b
                wait_gather(b)
                start_write(k, b)
                nk = k + NBUF

                @pl.when(nk < NCHUNK)
                def _():
                    wait_write(b)
                    start_gather(nk, b)

            return c2

        lax.fori_loop(0, NCHUNK // NBUF, ring, 0)
        for b in range(NBUF):
            wait_write(b)
        return c

    lax.fori_loop(0, BPW, do_batch, 0)


@jax.jit
def _permute_rows(xf, b1, b2):
    mesh = plsc.VectorSubcoreMesh(
        core_axis_name="c", subcore_axis_name="s", num_cores=NC, num_subcores=NS
    )
    return pl.kernel(
        _body,
        out_type=jax.ShapeDtypeStruct((ROWS, D), jnp.float32),
        mesh=mesh,
        scratch_types=(
            [pltpu.VMEM((N,), jnp.int32) for _ in range(4)]
            + [pltpu.VMEM((256 * L,), jnp.int32)]
            + [pltpu.VMEM((NCHUNK, C), jnp.int32)]
            + [pltpu.VMEM((C, D), jnp.float32) for _ in range(NBUF)]
            + [pltpu.SemaphoreType.DMA for _ in range(2 * NBUF)]
        ),
        compiler_params=pltpu.CompilerParams(needs_layout_passes=False),
        name="perm_radix_gather",
    )(xf, b1, b2)


def _bits(y):
    def one(yi):
        key = jax.random.PRNGKey(yi)
        k1, s1 = jax.random.split(key)
        b1 = jax.random.bits(s1, (N,), jnp.uint32)
        _, s2 = jax.random.split(k1)
        b2 = jax.random.bits(s2, (N,), jnp.uint32)
        return b1, b2

    b1, b2 = jax.vmap(one)(y)
    cast = lambda b: lax.bitcast_convert_type(b, jnp.int32)
    return cast(b1), cast(b2)


def kernel(x, y):
    b1, b2 = _bits(y)
    out = _permute_rows(x.reshape(ROWS, D), b1, b2)
    return out.reshape(BATCH, N, D)


# speedup vs baseline: 1.2428x; 1.1990x over previous
import functools

import jax
import jax.numpy as jnp
from jax import lax
from jax.experimental import pallas as pl
from jax.experimental.pallas import tpu as pltpu
from jax.experimental.pallas import tpu_sc as plsc

BATCH = 64
N = 4096
D = 128
NC = 2
NS = 16
NW = NC * NS
ROWS = BATCH * N
BPW = BATCH // NW
L = 16
VV = N // L
C = 128
NCHUNK = N // C
NBUF = 4


def _lane_iota():
    return lax.iota(jnp.int32, L)


def _digit(k, sh):
    if sh:
        k = lax.shift_right_logical(k, jnp.full((L,), sh, jnp.int32))
    return jnp.bitwise_and(k, jnp.full((L,), 255, jnp.int32))


def _radix_pass(ksrc, vsrc, kdst, vdst, hist, sh, first_round_pass0,
                val_off=None, store_keys=True):
    lanes = _lane_iota()

    @plsc.parallel_loop(0, 256 // 4)
    def zero(t):
        for u in range(4):
            hist[pl.ds((t * 4 + u) * L, L)] = jnp.zeros((L,), jnp.int32)

    @plsc.parallel_loop(0, VV, unroll=4)
    def histo(v):
        idx = lanes * VV + v
        k = plsc.load_gather(ksrc, [idx])
        h = _digit(k, sh) * L + lanes
        plsc.addupdate_scatter(hist, [h], jnp.ones((L,), jnp.int32))

    def scan(t, running):
        h0 = hist[pl.ds((t * 2) * L, L)]
        h1 = hist[pl.ds((t * 2 + 1) * L, L)]
        s0 = plsc.cumsum(h0)
        s1 = plsc.cumsum(h1)
        hist[pl.ds((t * 2) * L, L)] = s0 - h0 + running
        running = running + s0[L - 1]
        hist[pl.ds((t * 2 + 1) * L, L)] = s1 - h1 + running
        return running + s1[L - 1]

    lax.fori_loop(0, 256 // 2, scan, jnp.int32(0))

    def scatter(vh, c):
        for u in range(4):
            v = vh * 4 + u
            idx = lanes * VV + v
            k = plsc.load_gather(ksrc, [idx])
            if first_round_pass0:
                val = idx
            else:
                val = plsc.load_gather(vsrc, [idx])
            if val_off is not None:
                val = val + val_off
            h = _digit(k, sh) * L + lanes
            pos = plsc.load_gather(hist, [h])
            if store_keys:
                plsc.store_scatter(kdst, [pos], k)
            if len(vdst.shape) == 2:
                ph = lax.shift_right_logical(pos, jnp.full((L,), 7, jnp.int32))
                plsc.store_scatter(
                    vdst, [ph, jnp.bitwise_and(pos, jnp.full((L,), 127, jnp.int32))], val
                )
            else:
                plsc.store_scatter(vdst, [pos], val)
            plsc.store_scatter(hist, [h], pos + 1)
        return c

    lax.fori_loop(0, VV // 4, scatter, 0)


def _radix_argsort(ka, kb, va, vb, hist, first_round,
                   last_vdst=None, last_off=None):
    _radix_pass(ka, va, kb, vb, hist, 0, first_round)
    _radix_pass(kb, vb, ka, va, hist, 8, False)
    _radix_pass(ka, va, kb, vb, hist, 16, False)
    if last_vdst is None:
        _radix_pass(kb, vb, ka, va, hist, 24, False)
    else:
        _radix_pass(kb, vb, ka, last_vdst, hist, 24, False,
                    val_off=last_off, store_keys=False)


def _body(x_hbm, b1_hbm, b2_hbm, out_hbm, ka, kb, va, vb, hist, idx_v, *bufs):
    rows = bufs[:NBUF]
    gsem = bufs[NBUF : 2 * NBUF]
    wsem = bufs[2 * NBUF :]
    wid = lax.axis_index("s") * NC + lax.axis_index("c")

    def do_batch(bi, c):
        batch = wid * BPW + bi
        pltpu.sync_copy(b1_hbm.at[batch], ka)
        _radix_argsort(ka, kb, va, vb, hist, True)
        pltpu.sync_copy(b2_hbm.at[batch], ka)
        _radix_argsort(ka, kb, va, vb, hist, False,
                       last_vdst=idx_v, last_off=batch * N)

        obase = batch * N

        def start_gather(k, b):
            pltpu.async_copy(x_hbm.at[idx_v.at[k]], rows[b], gsem[b])

        def wait_gather(b):
            pltpu.make_async_copy(x_hbm.at[pl.ds(0, C)], rows[b], gsem[b]).wait()

        def start_write(k, b):
            pltpu.async_copy(rows[b], out_hbm.at[pl.ds(obase + k * C, C)], wsem[b])

        def wait_write(b):
            pltpu.make_async_copy(rows[b], out_hbm.at[pl.ds(0, C)], wsem[b]).wait()

        for b in range(NBUF):
            start_gather(b, b)

        def ring(i, c2):
            for b in range(NBUF):
                k = i * NBUF + b
                wait_gather(b)
                start_write(k, b)
                nk = k + NBUF

                @pl.when(nk < NCHUNK)
                def _():
                    wait_write(b)
                    start_gather(nk, b)

            return c2

        lax.fori_loop(0, NCHUNK // NBUF, ring, 0)
        for b in range(NBUF):
            wait_write(b)
        return c

    lax.fori_loop(0, BPW, do_batch, 0)


@jax.jit
def _permute_rows(xf, b1, b2):
    mesh = plsc.VectorSubcoreMesh(
        core_axis_name="c", subcore_axis_name="s", num_cores=NC, num_subcores=NS
    )
    return pl.kernel(
        _body,
        out_type=jax.ShapeDtypeStruct((ROWS, D), jnp.float32),
        mesh=mesh,
        scratch_types=(
            [pltpu.VMEM((N,), jnp.int32) for _ in range(4)]
            + [pltpu.VMEM((256 * L,), jnp.int32)]
            + [pltpu.VMEM((NCHUNK, C), jnp.int32)]
            + [pltpu.VMEM((C, D), jnp.float32) for _ in range(NBUF)]
            + [pltpu.SemaphoreType.DMA for _ in range(2 * NBUF)]
        ),
        compiler_params=pltpu.CompilerParams(needs_layout_passes=False),
        name="perm_radix_gather",
    )(xf, b1, b2)


def _bits(y):
    def one(yi):
        key = jax.random.PRNGKey(yi)
        k1, s1 = jax.random.split(key)
        b1 = jax.random.bits(s1, (N,), jnp.uint32)
        _, s2 = jax.random.split(k1)
        b2 = jax.random.bits(s2, (N,), jnp.uint32)
        return b1, b2

    b1, b2 = jax.vmap(one)(y)
    cast = lambda b: lax.bitcast_convert_type(b, jnp.int32)
    return cast(b1), cast(b2)


def kernel(x, y):
    b1, b2 = _bits(y)
    out = _permute_rows(x.reshape(ROWS, D), b1, b2)
    return out.reshape(BATCH, N, D)


# speedup vs baseline: 1.3999x; 1.1265x over previous
import functools

import jax
import jax.numpy as jnp
from jax import lax
from jax.experimental import pallas as pl
from jax.experimental.pallas import tpu as pltpu
from jax.experimental.pallas import tpu_sc as plsc

BATCH = 64
N = 4096
D = 128
NC = 2
NS = 16
NW = NC * NS
ROWS = BATCH * N
BPW = BATCH // NW
L = 16
VV = N // L
C = 128
NCHUNK = N // C
NBUF = 4


def _lane_iota():
    return lax.iota(jnp.int32, L)


def _digit(k, sh):
    if sh:
        k = lax.shift_right_logical(k, jnp.full((L,), sh, jnp.int32))
    return jnp.bitwise_and(k, jnp.full((L,), 255, jnp.int32))


def _radix_pass(ksrc, vsrc, kdst, vdst, hist, sh, first_round_pass0,
                val_off=None, store_keys=True, svc=(None, None)):
    lanes = _lane_iota()

    @plsc.parallel_loop(0, 256 // 4)
    def zero(t):
        for u in range(4):
            hist[pl.ds((t * 4 + u) * L, L)] = jnp.zeros((L,), jnp.int32)

    @plsc.parallel_loop(0, VV, unroll=4)
    def histo(v):
        idx = lanes * VV + v
        k = plsc.load_gather(ksrc, [idx])
        h = _digit(k, sh) * L + lanes
        plsc.addupdate_scatter(hist, [h], jnp.ones((L,), jnp.int32))

    if svc[0] is not None:
        svc[0]()

    def scan(t, running):
        h0 = hist[pl.ds((t * 2) * L, L)]
        h1 = hist[pl.ds((t * 2 + 1) * L, L)]
        s0 = plsc.cumsum(h0)
        s1 = plsc.cumsum(h1)
        hist[pl.ds((t * 2) * L, L)] = s0 - h0 + running
        running = running + s0[L - 1]
        hist[pl.ds((t * 2 + 1) * L, L)] = s1 - h1 + running
        return running + s1[L - 1]

    lax.fori_loop(0, 256 // 2, scan, jnp.int32(0))

    def scatter(vh, c):
        for u in range(4):
            v = vh * 4 + u
            idx = lanes * VV + v
            k = plsc.load_gather(ksrc, [idx])
            if first_round_pass0:
                val = idx
            else:
                val = plsc.load_gather(vsrc, [idx])
            if val_off is not None:
                val = val + val_off
            h = _digit(k, sh) * L + lanes
            pos = plsc.load_gather(hist, [h])
            if store_keys:
                plsc.store_scatter(kdst, [pos], k)
            if len(vdst.shape) == 2:
                ph = lax.shift_right_logical(pos, jnp.full((L,), 7, jnp.int32))
                plsc.store_scatter(
                    vdst, [ph, jnp.bitwise_and(pos, jnp.full((L,), 127, jnp.int32))], val
                )
            else:
                plsc.store_scatter(vdst, [pos], val)
            plsc.store_scatter(hist, [h], pos + 1)
        return c

    lax.fori_loop(0, VV // 4, scatter, 0)

    if svc[1] is not None:
        svc[1]()


def _radix_argsort(ka, kb, va, vb, hist, first_round,
                   last_vdst=None, last_off=None, svc4=None):
    if svc4 is None:
        svc4 = [(None, None)] * 4
    _radix_pass(ka, va, kb, vb, hist, 0, first_round, svc=svc4[0])
    _radix_pass(kb, vb, ka, va, hist, 8, False, svc=svc4[1])
    _radix_pass(ka, va, kb, vb, hist, 16, False, svc=svc4[2])
    if last_vdst is None:
        _radix_pass(kb, vb, ka, va, hist, 24, False, svc=svc4[3])
    else:
        _radix_pass(kb, vb, ka, last_vdst, hist, 24, False,
                    val_off=last_off, store_keys=False, svc=svc4[3])


def _body(x_hbm, b1_hbm, b2_hbm, out_hbm, ka, kb, va, vb, hist,
          idx_a, idx_b, ksem, *bufs):
    rows = bufs[:NBUF]
    gsem = bufs[NBUF : 2 * NBUF]
    wsem = bufs[2 * NBUF :]
    wid = lax.axis_index("s") * NC + lax.axis_index("c")

    def start_gather(idx_ref, k, b):
        pltpu.async_copy(x_hbm.at[idx_ref.at[k]], rows[b], gsem[b])

    def wait_gather(b):
        pltpu.make_async_copy(x_hbm.at[pl.ds(0, C)], rows[b], gsem[b]).wait()

    def start_write(obase, k, b):
        pltpu.async_copy(rows[b], out_hbm.at[pl.ds(obase + k * C, C)], wsem[b])

    def wait_write(b):
        pltpu.make_async_copy(rows[b], out_hbm.at[pl.ds(0, C)], wsem[b]).wait()

    def sort_batch(batch, idx_ref, svc4=None):
        pltpu.async_copy(b1_hbm.at[batch], ka, ksem).wait()
        _radix_argsort(ka, kb, va, vb, hist, True,
                       svc4=svc4[:4] if svc4 else None)
        pltpu.async_copy(b2_hbm.at[batch], ka, ksem).wait()
        _radix_argsort(ka, kb, va, vb, hist, False,
                       last_vdst=idx_ref, last_off=batch * N,
                       svc4=svc4[4:] if svc4 else None)

    def ring_step(idx_ref, obase, k):
        b = k % NBUF
        wait_gather(b)
        start_write(obase, k, b)
        nk = k + 2
        if nk < NCHUNK:
            if nk >= NBUF:
                wait_write(nk % NBUF)
            start_gather(idx_ref, nk, nk % NBUF)

    def plain_ring(idx_ref, obase):
        for b in range(NBUF):
            start_gather(idx_ref, b, b)

        def ring(i, c2):
            for b in range(NBUF):
                k = i * NBUF + b
                wait_gather(b)
                start_write(obase, k, b)
                nk = k + NBUF

                @pl.when(nk < NCHUNK)
                def _():
                    wait_write(b)
                    start_gather(idx_ref, nk, b)

            return c2

        lax.fori_loop(0, NCHUNK // NBUF, ring, 0)
        for b in range(NBUF):
            wait_write(b)

    batch_a = wid * BPW
    batch_b = batch_a + 1
    obase_a = batch_a * N
    obase_b = batch_b * N

    sort_batch(batch_a, idx_a)

    def mk_svc(q):
        def thunk():
            if q == 0:
                for b in range(2):
                    start_gather(idx_a, b, b)
            else:
                ring_step(idx_a, obase_a, 2 * (q - 1))
                ring_step(idx_a, obase_a, 2 * (q - 1) + 1)
        return thunk

    svc16 = [(mk_svc(2 * p), mk_svc(2 * p + 1)) for p in range(8)]
    sort_batch(batch_b, idx_b, svc4=svc16)
    ring_step(idx_a, obase_a, NCHUNK - 2)
    ring_step(idx_a, obase_a, NCHUNK - 1)
    for b in range(NBUF):
        wait_write(b)
    plain_ring(idx_b, obase_b)


@jax.jit
def _permute_rows(xf, b1, b2):
    mesh = plsc.VectorSubcoreMesh(
        core_axis_name="c", subcore_axis_name="s", num_cores=NC, num_subcores=NS
    )
    return pl.kernel(
        _body,
        out_type=jax.ShapeDtypeStruct((ROWS, D), jnp.float32),
        mesh=mesh,
        scratch_types=(
            [pltpu.VMEM((N,), jnp.int32) for _ in range(4)]
            + [pltpu.VMEM((256 * L,), jnp.int32)]
            + [pltpu.VMEM((NCHUNK, C), jnp.int32)]
            + [pltpu.VMEM((NCHUNK, C), jnp.int32)]
            + [pltpu.SemaphoreType.DMA]
            + [pltpu.VMEM((C, D), jnp.float32) for _ in range(NBUF)]
            + [pltpu.SemaphoreType.DMA for _ in range(2 * NBUF)]
        ),
        compiler_params=pltpu.CompilerParams(needs_layout_passes=False),
        name="perm_radix_gather",
    )(xf, b1, b2)


def _bits(y):
    def one(yi):
        key = jax.random.PRNGKey(yi)
        k1, s1 = jax.random.split(key)
        b1 = jax.random.bits(s1, (N,), jnp.uint32)
        _, s2 = jax.random.split(k1)
        b2 = jax.random.bits(s2, (N,), jnp.uint32)
        return b1, b2

    b1, b2 = jax.vmap(one)(y)
    cast = lambda b: lax.bitcast_convert_type(b, jnp.int32)
    return cast(b1), cast(b2)


def kernel(x, y):
    b1, b2 = _bits(y)
    out = _permute_rows(x.reshape(ROWS, D), b1, b2)
    return out.reshape(BATCH, N, D)
